# Initial kernel scaffold; baseline (speedup 1.0000x reference)
import jax
import jax.numpy as jnp
from jax.experimental import pallas as pl


def kernel(y, p, weights, edge):
    raise NotImplementedError("write your pallas kernel here")


# speedup vs baseline: 6.0887x; 6.0887x over previous
import jax
import jax.numpy as jnp
from jax.experimental import pallas as pl
from jax.experimental.pallas import tpu as pltpu


def _wmae_body(w_ref, e_ref, y_ref, p_ref, out_ref):
    j = pl.program_id(1)

    y = y_ref[0]
    d = jnp.abs(p_ref[0] - y)
    w = jnp.zeros_like(y)
    for b in range(w_ref.shape[0]):
        w = jnp.where(y >= e_ref[b], w_ref[b], w)
    below_top = y < e_ref[e_ref.shape[0] - 1]
    w = jnp.where(below_top, w, 0.0)
    valid = jnp.where((y >= e_ref[0]) & below_top, 1.0, 0.0)

    ps = jnp.sum(w * d)
    pc = jnp.sum(valid)

    @pl.when(j == 0)
    def _():
        out_ref[0, 0, 0] = ps
        out_ref[0, 0, 1] = pc

    @pl.when(j > 0)
    def _():
        out_ref[0, 0, 0] += ps
        out_ref[0, 0, 1] += pc


def kernel(y, p, weights, edge):
    b, h, w = y.shape
    ncores = 2
    per = b // ncores
    partials = pl.pallas_call(
        _wmae_body,
        grid=(ncores, per),
        in_specs=[
            pl.BlockSpec(memory_space=pltpu.SMEM),
            pl.BlockSpec(memory_space=pltpu.SMEM),
            pl.BlockSpec((1, h, w), lambda i, j: (i * per + j, 0, 0)),
            pl.BlockSpec((1, h, w), lambda i, j: (i * per + j, 0, 0)),
        ],
        out_specs=pl.BlockSpec((1, 1, 2), lambda i, j: (i, 0, 0),
                               memory_space=pltpu.SMEM),
        out_shape=jax.ShapeDtypeStruct((ncores, 1, 2), jnp.float32),
        compiler_params=pltpu.CompilerParams(
            dimension_semantics=("parallel", "arbitrary")),
    )(weights, edge, y, p)
    return partials[:, 0, 0].sum() / partials[:, 0, 1].sum()


# speedup vs baseline: 7.7618x; 1.2748x over previous
import jax
import jax.numpy as jnp
from jax.experimental import pallas as pl
from jax.experimental.pallas import tpu as pltpu


def _wmae_body(w_ref, e_ref, y_ref, p_ref, out_ref):
    j = pl.program_id(1)

    ps = jnp.float32(0.0)
    pc = jnp.float32(0.0)
    for k in range(y_ref.shape[0]):
        y = y_ref[k]
        d = jnp.abs(p_ref[k] - y)
        w = jnp.zeros_like(y)
        for b in range(w_ref.shape[0]):
            w = jnp.where(y >= e_ref[b], w_ref[b], w)
        below_top = y < e_ref[e_ref.shape[0] - 1]
        w = jnp.where(below_top, w, 0.0)
        valid = jnp.where((y >= e_ref[0]) & below_top, 1.0, 0.0)
        ps = ps + jnp.sum(w * d)
        pc = pc + jnp.sum(valid)

    @pl.when(j == 0)
    def _():
        out_ref[0, 0, 0] = ps
        out_ref[0, 0, 1] = pc

    @pl.when(j > 0)
    def _():
        out_ref[0, 0, 0] += ps
        out_ref[0, 0, 1] += pc


def kernel(y, p, weights, edge):
    b, h, w = y.shape
    ncores = 2
    bb = 4
    per = b // (ncores * bb)
    partials = pl.pallas_call(
        _wmae_body,
        grid=(ncores, per),
        in_specs=[
            pl.BlockSpec(memory_space=pltpu.SMEM),
            pl.BlockSpec(memory_space=pltpu.SMEM),
            pl.BlockSpec((bb, h, w), lambda i, j: (i * per + j, 0, 0)),
            pl.BlockSpec((bb, h, w), lambda i, j: (i * per + j, 0, 0)),
        ],
        out_specs=pl.BlockSpec((1, 1, 2), lambda i, j: (i, 0, 0),
                               memory_space=pltpu.SMEM),
        out_shape=jax.ShapeDtypeStruct((ncores, 1, 2), jnp.float32),
        compiler_params=pltpu.CompilerParams(
            dimension_semantics=("parallel", "arbitrary")),
    )(weights, edge, y, p)
    return partials[:, 0, 0].sum() / partials[:, 0, 1].sum()


# speedup vs baseline: 9.6854x; 1.2478x over previous
import jax
import jax.numpy as jnp
from jax.experimental import pallas as pl
from jax.experimental.pallas import tpu as pltpu


def _wmae_body(w_ref, e_ref, y_ref, p_ref, out_ref):
    j = pl.program_id(1)

    nb, h, wd_ = y_ref.shape
    rows = 32
    acc_s = jnp.zeros((8, wd_), jnp.float32)
    acc_c = jnp.zeros((8, wd_), jnp.float32)
    for k in range(nb):
        for c in range(h // rows):
            y = y_ref[k, c * rows:(c + 1) * rows, :]
            d = jnp.abs(p_ref[k, c * rows:(c + 1) * rows, :] - y)
            w = jnp.zeros_like(y)
            for b in range(w_ref.shape[0]):
                w = jnp.where(y >= e_ref[b], w_ref[b], w)
            below_top = y < e_ref[e_ref.shape[0] - 1]
            w = jnp.where(below_top, w, 0.0)
            valid = jnp.where((y >= e_ref[0]) & below_top, 1.0, 0.0)
            wd = w * d
            for r in range(rows // 8):
                acc_s = acc_s + wd[r * 8:(r + 1) * 8, :]
                acc_c = acc_c + valid[r * 8:(r + 1) * 8, :]
    ps = jnp.sum(acc_s)
    pc = jnp.sum(acc_c)

    @pl.when(j == 0)
    def _():
        out_ref[0, 0, 0] = ps
        out_ref[0, 0, 1] = pc

    @pl.when(j > 0)
    def _():
        out_ref[0, 0, 0] += ps
        out_ref[0, 0, 1] += pc


def kernel(y, p, weights, edge):
    b, h, w = y.shape
    ncores = 2
    bb = 4
    per = b // (ncores * bb)
    partials = pl.pallas_call(
        _wmae_body,
        grid=(ncores, per),
        in_specs=[
            pl.BlockSpec(memory_space=pltpu.SMEM),
            pl.BlockSpec(memory_space=pltpu.SMEM),
            pl.BlockSpec((bb, h, w), lambda i, j: (i * per + j, 0, 0)),
            pl.BlockSpec((bb, h, w), lambda i, j: (i * per + j, 0, 0)),
        ],
        out_specs=pl.BlockSpec((1, 1, 2), lambda i, j: (i, 0, 0),
                               memory_space=pltpu.SMEM),
        out_shape=jax.ShapeDtypeStruct((ncores, 1, 2), jnp.float32),
        compiler_params=pltpu.CompilerParams(
            dimension_semantics=("parallel", "arbitrary")),
    )(weights, edge, y, p)
    return partials[:, 0, 0].sum() / partials[:, 0, 1].sum()
